# Initial kernel scaffold; baseline (speedup 1.0000x reference)
import jax
import jax.numpy as jnp
from jax.experimental import pallas as pl


def kernel(x, mask, leavesList, ancestorsList, W_emb, W_attention, b_attention, v_attention, W_ih, W_hh, b_ih, b_hh, W_output, b_output):
    raise NotImplementedError("write your pallas kernel here")


# speedup vs baseline: 1.2012x; 1.2012x over previous
import functools

import jax
import jax.numpy as jnp
from jax import lax
from jax.experimental import pallas as pl
from jax.experimental.pallas import tpu as pltpu
from jax.experimental.pallas import tpu_sc as plsc

_NC = 2
_NS = 16
_NW = _NC * _NS
_LANES = 16
_GROUP = 8


def _pq_project(W_emb, Wcat, bcat):
    Vtab = W_emb.shape[0]
    D2 = Wcat.shape[1]

    def body(wemb_ref, wcat_ref, b_ref, pq_ref):
        pq_ref[...] = (
            jnp.dot(wemb_ref[...], wcat_ref[...], preferred_element_type=jnp.float32)
            + b_ref[...]
        )

    return pl.pallas_call(
        body,
        out_shape=jax.ShapeDtypeStruct((Vtab, D2), jnp.float32),
    )(W_emb, Wcat, bcat.reshape(1, D2))


def _tanh16(s):
    return 2.0 / (1.0 + jnp.exp(-2.0 * s)) - 1.0


def _attention_emb(PQ, W_emb, v_att, lidx, aidx, vpad, A, D, E):
    cpt = vpad // _NW
    ngroups = cpt // _GROUP
    rows = _GROUP * A
    mesh = plsc.VectorSubcoreMesh(core_axis_name="c", subcore_axis_name="s")

    @functools.partial(
        pl.kernel,
        out_type=jax.ShapeDtypeStruct((vpad, E), jnp.float32),
        mesh=mesh,
        compiler_params=pltpu.CompilerParams(needs_layout_passes=False),
        scratch_types=[
            pltpu.VMEM((ngroups, rows), jnp.int32),
            pltpu.VMEM((ngroups, rows), jnp.int32),
            pltpu.VMEM((rows, 2 * D), jnp.float32),
            pltpu.VMEM((rows, 2 * D), jnp.float32),
            pltpu.VMEM((rows, E), jnp.float32),
            pltpu.VMEM((cpt, E), jnp.float32),
            pltpu.VMEM((A * A,), jnp.float32),
            pltpu.VMEM((A,), jnp.float32),
            pltpu.VMEM((D,), jnp.float32),
            pltpu.SemaphoreType.DMA,
            pltpu.SemaphoreType.DMA,
            pltpu.SemaphoreType.DMA,
        ],
    )
    def attn(pq_hbm, wemb_hbm, vatt_hbm, lidx_hbm, aidx_hbm, out_hbm,
             lidx_v, aidx_v, pl_v, qa_v, anc_v, emb_v, r_v, att_v, vatt_v,
             sem0, sem1, sem2):
        wid = lax.axis_index("s") * _NC + lax.axis_index("c")
        pltpu.sync_copy(lidx_hbm.at[wid], lidx_v)
        pltpu.sync_copy(aidx_hbm.at[wid], aidx_v)
        pltpu.sync_copy(vatt_hbm, vatt_v)
        lane_iota = lax.iota(jnp.int32, _LANES)

        def group_body(g, _):
            d0 = pltpu.async_copy(pq_hbm.at[lidx_v.at[g]], pl_v, sem0)
            d1 = pltpu.async_copy(pq_hbm.at[aidx_v.at[g]], qa_v, sem1)
            d2 = pltpu.async_copy(wemb_hbm.at[aidx_v.at[g]], anc_v, sem2)
            d0.wait()
            d1.wait()
            d2.wait()

            def code_body(c, _):
                row0 = c * A

                def mlp_body(a, _):
                    r = row0 + a
                    acc = jnp.zeros((_LANES,), jnp.float32)
                    for j in range(D // _LANES):
                        s = pl_v[r, pl.ds(j * _LANES, _LANES)] + qa_v[
                            r, pl.ds(D + j * _LANES, _LANES)
                        ]
                        acc = acc + _tanh16(s) * vatt_v[pl.ds(j * _LANES, _LANES)]
                    r_v[pl.ds(a * _LANES, _LANES)] = acc
                    return 0

                lax.fori_loop(0, A, mlp_body, 0, unroll=False)

                pre = jnp.zeros((_LANES,), jnp.float32)
                for l in range(A):
                    pre = pre + plsc.load_gather(r_v, [lane_iota * A + l])
                m = jnp.max(pre)
                e = jnp.exp(pre - m)
                att_v[...] = e / jnp.sum(e)

                def wsum_body(a, accs):
                    w = plsc.load_gather(att_v, [jnp.full((_LANES,), a, jnp.int32)])
                    r = row0 + a
                    return tuple(
                        accs[j] + w * anc_v[r, pl.ds(j * _LANES, _LANES)]
                        for j in range(E // _LANES)
                    )

                accs = lax.fori_loop(
                    0, A, wsum_body,
                    tuple(jnp.zeros((_LANES,), jnp.float32) for _ in range(E // _LANES)),
                    unroll=False,
                )
                code = g * _GROUP + c
                for j in range(E // _LANES):
                    emb_v[code, pl.ds(j * _LANES, _LANES)] = accs[j]
                return 0

            lax.fori_loop(0, _GROUP, code_body, 0, unroll=False)
            return 0

        lax.fori_loop(0, ngroups, group_body, 0, unroll=False)
        pltpu.sync_copy(emb_v, out_hbm.at[pl.ds(wid * cpt, cpt)])

    return attn(PQ, W_emb, v_att, lidx, aidx)


def _sequence(emb, x, mask, W_ihT, W_hhT, b_ih, b_hh, W_output, b_output):
    T, B, V = x.shape
    E = emb.shape[1]
    H = W_hhT.shape[0]
    C = W_output.shape[1]
    mask3 = mask[:, :, None]

    def body(emb_ref, x_ref, m_ref, wih_ref, whh_ref, bih_ref, bhh_ref,
             wout_ref, bout_ref, y_ref, h_ref):
        t = pl.program_id(0)

        @pl.when(t == 0)
        def _():
            h_ref[...] = jnp.zeros_like(h_ref)

        xe = jnp.tanh(
            jnp.dot(x_ref[0], emb_ref[...], preferred_element_type=jnp.float32)
        )
        h = h_ref[...]
        gi = jnp.dot(xe, wih_ref[...], preferred_element_type=jnp.float32) + bih_ref[...]
        gh = jnp.dot(h, whh_ref[...], preferred_element_type=jnp.float32) + bhh_ref[...]
        r = jax.nn.sigmoid(gi[:, :H] + gh[:, :H])
        z = jax.nn.sigmoid(gi[:, H:2 * H] + gh[:, H:2 * H])
        n = jnp.tanh(gi[:, 2 * H:] + r * gh[:, 2 * H:])
        hnew = (1.0 - z) * n + z * h
        h_ref[...] = hnew
        nom = jnp.exp(
            jnp.dot(hnew, wout_ref[...], preferred_element_type=jnp.float32)
            + bout_ref[...]
        )
        y_ref[0] = nom / jnp.sum(nom, axis=1, keepdims=True) * m_ref[0]

    full = lambda shape: pl.BlockSpec(shape, lambda t: tuple(0 for _ in shape))
    return pl.pallas_call(
        body,
        grid=(T,),
        in_specs=[
            full(emb.shape),
            pl.BlockSpec((1, B, V), lambda t: (t, 0, 0)),
            pl.BlockSpec((1, B, 1), lambda t: (t, 0, 0)),
            full(W_ihT.shape),
            full(W_hhT.shape),
            full((1, 3 * H)),
            full((1, 3 * H)),
            full(W_output.shape),
            full((1, C)),
        ],
        out_specs=pl.BlockSpec((1, B, C), lambda t: (t, 0, 0)),
        out_shape=jax.ShapeDtypeStruct((T, B, C), jnp.float32),
        scratch_shapes=[pltpu.VMEM((B, H), jnp.float32)],
    )(emb, x, mask3, W_ihT, W_hhT, b_ih.reshape(1, 3 * H),
      b_hh.reshape(1, 3 * H), W_output, b_output.reshape(1, C))


def kernel(x, mask, leavesList, ancestorsList, W_emb, W_attention, b_attention,
           v_attention, W_ih, W_hh, b_ih, b_hh, W_output, b_output):
    T, B, V = x.shape
    A = leavesList.shape[2]
    E = W_emb.shape[1]
    D = W_attention.shape[1]

    Wcat = jnp.concatenate([W_attention[:E], W_attention[E:]], axis=1)
    bcat = jnp.concatenate([b_attention, jnp.zeros_like(b_attention)])
    PQ = _pq_project(W_emb, Wcat, bcat)

    vpad = -(-V // (_NW * _GROUP)) * (_NW * _GROUP)
    lidx = leavesList.reshape(V, A).astype(jnp.int32)
    aidx = ancestorsList.reshape(V, A).astype(jnp.int32)
    pad = vpad - V
    if pad:
        lidx = jnp.pad(lidx, ((0, pad), (0, 0)))
        aidx = jnp.pad(aidx, ((0, pad), (0, 0)))
    rows = _GROUP * A
    lidx = lidx.reshape(_NW, vpad // (_NW * _GROUP), rows)
    aidx = aidx.reshape(_NW, vpad // (_NW * _GROUP), rows)

    emb = _attention_emb(PQ, W_emb, v_attention, lidx, aidx, vpad, A, D, E)[:V]

    return _sequence(emb, x, mask, W_ih.T, W_hh.T, b_ih, b_hh, W_output, b_output)


# speedup vs baseline: 1.5902x; 1.3239x over previous
import functools

import jax
import jax.numpy as jnp
from jax import lax
from jax.experimental import pallas as pl
from jax.experimental.pallas import tpu as pltpu
from jax.experimental.pallas import tpu_sc as plsc

_NC = 2
_NS = 16
_NW = _NC * _NS
_LANES = 16
_GROUP = 8


def _pq_project(W_emb, W1, W2, b_att):
    Vtab = W_emb.shape[0]
    D = W1.shape[1]

    def body(wemb_ref, w1_ref, w2_ref, b_ref, tl_ref, ta_ref):
        w = wemb_ref[...]
        p = jnp.dot(w, w1_ref[...], preferred_element_type=jnp.float32) + b_ref[...]
        q = jnp.dot(w, w2_ref[...], preferred_element_type=jnp.float32)
        tl_ref[...] = jnp.concatenate([-2.0 * p, jnp.zeros_like(p)], axis=1)
        ta_ref[...] = jnp.concatenate([-2.0 * q, jnp.zeros_like(q)], axis=1)

    return pl.pallas_call(
        body,
        out_shape=[
            jax.ShapeDtypeStruct((Vtab, 2 * D), jnp.float32),
            jax.ShapeDtypeStruct((Vtab, 2 * D), jnp.float32),
        ],
    )(W_emb, W1, W2, b_att.reshape(1, D))


def _attention_emb(Tl, Ta, W_emb, v_att, lidx, aidx, vpad, A, D, E):
    cpt = vpad // _NW
    ngroups = cpt // _GROUP
    rows = _GROUP * A
    mesh = plsc.VectorSubcoreMesh(core_axis_name="c", subcore_axis_name="s")

    @functools.partial(
        pl.kernel,
        out_type=jax.ShapeDtypeStruct((vpad, E), jnp.float32),
        mesh=mesh,
        compiler_params=pltpu.CompilerParams(needs_layout_passes=False),
        scratch_types=[
            pltpu.VMEM((ngroups, rows), jnp.int32),
            pltpu.VMEM((ngroups, rows), jnp.int32),
            pltpu.VMEM((rows, 2 * D), jnp.float32),
            pltpu.VMEM((rows, 2 * D), jnp.float32),
            pltpu.VMEM((rows, E), jnp.float32),
            pltpu.VMEM((rows, E), jnp.float32),
            pltpu.VMEM((cpt, E), jnp.float32),
            pltpu.VMEM((A * A,), jnp.float32),
            pltpu.VMEM((A,), jnp.float32),
            pltpu.VMEM((D,), jnp.float32),
            pltpu.SemaphoreType.DMA,
            pltpu.SemaphoreType.DMA,
            pltpu.SemaphoreType.DMA,
            pltpu.SemaphoreType.DMA,
        ],
    )
    def attn(tl_hbm, ta_hbm, wemb_hbm, vatt_hbm, lidx_hbm, aidx_hbm, out_hbm,
             lidx_v, aidx_v, s_v0, s_v1, anc_v0, anc_v1, emb_v, r_v, att_v,
             vatt_v, sem_s0, sem_s1, sem_a0, sem_a1):
        wid = lax.axis_index("s") * _NC + lax.axis_index("c")
        pltpu.sync_copy(lidx_hbm.at[wid], lidx_v)
        pltpu.sync_copy(aidx_hbm.at[wid], aidx_v)
        pltpu.sync_copy(vatt_hbm, vatt_v)
        for j in range(D // _LANES):
            sl = pl.ds(j * _LANES, _LANES)
            vatt_v[sl] = vatt_v[sl] * 2.0
        lane_iota = lax.iota(jnp.int32, _LANES)
        s_bufs = (s_v0, s_v1)
        anc_bufs = (anc_v0, anc_v1)
        s_sems = (sem_s0, sem_s1)
        a_sems = (sem_a0, sem_a1)

        def issue_base(g, slot):
            pltpu.async_copy(tl_hbm.at[lidx_v.at[g]], s_bufs[slot], s_sems[slot])
            pltpu.async_copy(wemb_hbm.at[aidx_v.at[g]], anc_bufs[slot], a_sems[slot])

        def wait_s(g, slot):
            pltpu.make_async_copy(
                tl_hbm.at[lidx_v.at[g]], s_bufs[slot], s_sems[slot]
            ).wait()

        def wait_a(g, slot):
            pltpu.make_async_copy(
                wemb_hbm.at[aidx_v.at[g]], anc_bufs[slot], a_sems[slot]
            ).wait()

        def compute(g, slot):
            s_v = s_bufs[slot]
            anc_v = anc_bufs[slot]

            def code_body(c, _):
                row0 = c * A

                def mlp_body(a, _):
                    r = row0 + a
                    acc = jnp.zeros((_LANES,), jnp.float32)
                    for j in range(D // _LANES):
                        e = jnp.exp(s_v[r, pl.ds(j * _LANES, _LANES)])
                        acc = acc + vatt_v[pl.ds(j * _LANES, _LANES)] / (1.0 + e)
                    r_v[pl.ds(a * _LANES, _LANES)] = acc
                    return 0

                lax.fori_loop(0, A, mlp_body, 0, unroll=False)

                pre = jnp.zeros((_LANES,), jnp.float32)
                for l in range(A):
                    pre = pre + plsc.load_gather(r_v, [lane_iota * A + l])
                m = jnp.max(pre)
                e = jnp.exp(pre - m)
                att_v[...] = e / jnp.sum(e)

                def wsum_body(a, accs):
                    w = plsc.load_gather(att_v, [jnp.full((_LANES,), a, jnp.int32)])
                    r = row0 + a
                    return tuple(
                        accs[j] + w * anc_v[r, pl.ds(j * _LANES, _LANES)]
                        for j in range(E // _LANES)
                    )

                accs = lax.fori_loop(
                    0, A, wsum_body,
                    tuple(jnp.zeros((_LANES,), jnp.float32) for _ in range(E // _LANES)),
                    unroll=False,
                )
                code = g * _GROUP + c
                for j in range(E // _LANES):
                    emb_v[code, pl.ds(j * _LANES, _LANES)] = accs[j]
                return 0

            lax.fori_loop(0, _GROUP, code_body, 0, unroll=False)

        issue_base(0, 0)

        def pair_body(i, _):
            g0 = 2 * i
            g1 = g0 + 1
            wait_s(g0, 0)
            pltpu.async_copy(ta_hbm.at[aidx_v.at[g0]], s_v0, sem_s0, add=True)
            issue_base(g1, 1)
            wait_s(g0, 0)
            wait_a(g0, 0)
            compute(g0, 0)
            wait_s(g1, 1)
            pltpu.async_copy(ta_hbm.at[aidx_v.at[g1]], s_v1, sem_s1, add=True)

            @pl.when(g1 + 1 < ngroups)
            def _():
                issue_base(g1 + 1, 0)

            wait_s(g1, 1)
            wait_a(g1, 1)
            compute(g1, 1)
            return 0

        lax.fori_loop(0, ngroups // 2, pair_body, 0, unroll=False)
        pltpu.sync_copy(emb_v, out_hbm.at[pl.ds(wid * cpt, cpt)])

    return attn(Tl, Ta, W_emb, v_att, lidx, aidx)


def _sequence(emb, x, mask, W_ihT, W_hhT, b_ih, b_hh, W_output, b_output):
    T, B, V = x.shape
    E = emb.shape[1]
    H = W_hhT.shape[0]
    C = W_output.shape[1]
    mask3 = mask[:, :, None]

    def body(emb_ref, x_ref, m_ref, wih_ref, whh_ref, bih_ref, bhh_ref,
             wout_ref, bout_ref, y_ref, h_ref):
        t = pl.program_id(0)

        @pl.when(t == 0)
        def _():
            h_ref[...] = jnp.zeros_like(h_ref)

        xe = jnp.tanh(
            jnp.dot(x_ref[0], emb_ref[...], preferred_element_type=jnp.float32)
        )
        h = h_ref[...]
        gi = jnp.dot(xe, wih_ref[...], preferred_element_type=jnp.float32) + bih_ref[...]
        gh = jnp.dot(h, whh_ref[...], preferred_element_type=jnp.float32) + bhh_ref[...]
        r = jax.nn.sigmoid(gi[:, :H] + gh[:, :H])
        z = jax.nn.sigmoid(gi[:, H:2 * H] + gh[:, H:2 * H])
        n = jnp.tanh(gi[:, 2 * H:] + r * gh[:, 2 * H:])
        hnew = (1.0 - z) * n + z * h
        h_ref[...] = hnew
        nom = jnp.exp(
            jnp.dot(hnew, wout_ref[...], preferred_element_type=jnp.float32)
            + bout_ref[...]
        )
        y_ref[0] = nom / jnp.sum(nom, axis=1, keepdims=True) * m_ref[0]

    full = lambda shape: pl.BlockSpec(shape, lambda t: tuple(0 for _ in shape))
    return pl.pallas_call(
        body,
        grid=(T,),
        in_specs=[
            full(emb.shape),
            pl.BlockSpec((1, B, V), lambda t: (t, 0, 0)),
            pl.BlockSpec((1, B, 1), lambda t: (t, 0, 0)),
            full(W_ihT.shape),
            full(W_hhT.shape),
            full((1, 3 * H)),
            full((1, 3 * H)),
            full(W_output.shape),
            full((1, C)),
        ],
        out_specs=pl.BlockSpec((1, B, C), lambda t: (t, 0, 0)),
        out_shape=jax.ShapeDtypeStruct((T, B, C), jnp.float32),
        scratch_shapes=[pltpu.VMEM((B, H), jnp.float32)],
    )(emb, x, mask3, W_ihT, W_hhT, b_ih.reshape(1, 3 * H),
      b_hh.reshape(1, 3 * H), W_output, b_output.reshape(1, C))


def kernel(x, mask, leavesList, ancestorsList, W_emb, W_attention, b_attention,
           v_attention, W_ih, W_hh, b_ih, b_hh, W_output, b_output):
    T, B, V = x.shape
    A = leavesList.shape[2]
    E = W_emb.shape[1]
    D = W_attention.shape[1]

    Tl, Ta = _pq_project(W_emb, W_attention[:E], W_attention[E:], b_attention)

    vpad = -(-V // (_NW * _GROUP)) * (_NW * _GROUP)
    lidx = leavesList.reshape(V, A).astype(jnp.int32)
    aidx = ancestorsList.reshape(V, A).astype(jnp.int32)
    pad = vpad - V
    if pad:
        lidx = jnp.pad(lidx, ((0, pad), (0, 0)))
        aidx = jnp.pad(aidx, ((0, pad), (0, 0)))
    rows = _GROUP * A
    lidx = lidx.reshape(_NW, vpad // (_NW * _GROUP), rows)
    aidx = aidx.reshape(_NW, vpad // (_NW * _GROUP), rows)

    emb = _attention_emb(Tl, Ta, W_emb, v_attention, lidx, aidx, vpad, A, D, E)[:V]

    return _sequence(emb, x, mask, W_ih.T, W_hh.T, b_ih, b_hh, W_output, b_output)
